# Initial kernel scaffold; baseline (speedup 1.0000x reference)
import jax
import jax.numpy as jnp
from jax.experimental import pallas as pl


def kernel(x, pos, gnn_W, gnn_b, mlp_W1, mlp_b1, mlp_W2, mlp_b2, edge_index, batch):
    raise NotImplementedError("write your pallas kernel here")


# speedup vs baseline: 2.8439x; 2.8439x over previous
import functools

import jax
import jax.numpy as jnp
from jax.experimental import pallas as pl
from jax.experimental.pallas import tpu as pltpu
from jax.experimental.pallas import tpu_sc as plsc

N = 2048
E = 32768
D = 128
L = 3
G = 16
BFS_ITERS = 6
BLK = 256
NBLK = N // BLK

_HIGH = jax.lax.Precision.HIGHEST


def _dot(a, b):
    return jnp.dot(a, b, precision=_HIGH, preferred_element_type=jnp.float32)


def _fused_body(mt_ref, x_ref, gw_ref, gb_ref, w1_ref, b1_ref, w2_ref,
                b2_ref, batch_ref, pos_ref,
                pred_ref, gf_ref, ploss_ref, mani_ref,
                h_ref, t_ref, a_ref):
    h_ref[...] = x_ref[...]
    for l in range(L):
        def blk(b, carry, l=l):
            rows = pl.ds(b * BLK, BLK)
            mtb = mt_ref[rows, :]
            if l == 0:
                a_ref[rows, :] = (mtb > 0.0).astype(jnp.bfloat16)
            aggb = _dot(mtb, h_ref[...])
            degb = jnp.sum(mtb, axis=1, keepdims=True) + 1.0
            t_ref[rows, :] = (h_ref[rows, :] + aggb) / degb
            return carry

        jax.lax.fori_loop(0, NBLK, blk, 0)
        hn = _dot(t_ref[...], gw_ref[l]) + gb_ref[l:l + 1, :]
        if l < L - 1:
            hn = jnp.maximum(hn, 0.0)
        h_ref[...] = hn
    h = h_ref[...]
    bb = jnp.broadcast_to(batch_ref[...], (G, N))
    gi = jax.lax.broadcasted_iota(jnp.int32, (G, N), 0)
    p = (bb == gi).astype(jnp.float32)
    counts = jnp.sum(p, axis=1, keepdims=True)
    gf_ref[...] = _dot(p, h) / jnp.maximum(counts, 1.0)
    t = jnp.maximum(_dot(h, w1_ref[...]) + b1_ref[...], 0.0)
    pred = _dot(t, w2_ref[...]) + b2_ref[...]
    pred_ref[...] = pred
    d = pred - pos_ref[...]
    ploss_ref[...] = (jnp.sum(d * d) * (1.0 / (N * 3))).reshape(1, 1)
    predt = jax.lax.transpose(pred, (1, 0))

    one = jnp.bfloat16(1.0)
    inf = jnp.bfloat16(jnp.inf)

    def build(b, carry):
        base = b * BLK
        ubt_b = jax.lax.transpose(a_ref[:, pl.ds(base, BLK)], (1, 0))
        a_ref[pl.ds(base, BLK), :] = jnp.maximum(
            a_ref[pl.ds(base, BLK), :], ubt_b)
        return carry

    jax.lax.fori_loop(0, NBLK, build, 0)
    a = a_ref[...]

    def block_step(b, acc):
        base = b * BLK
        ii = jax.lax.broadcasted_iota(jnp.int32, (BLK, N), 0) + base
        jj = jax.lax.broadcasted_iota(jnp.int32, (BLK, N), 1)
        eye = ii == jj
        ab = a_ref[pl.ds(base, BLK), :]
        r = jnp.where(eye, one, ab)
        dist = jnp.where(eye, jnp.bfloat16(0.0), jnp.where(ab > 0, one, inf))
        for k in range(2, BFS_ITERS + 1):
            prod = jnp.dot(r, a, preferred_element_type=jnp.float32)
            r = jnp.where(prod > 0.0, one, r)
            dist = jnp.where((r > 0) & (dist == inf),
                             jnp.bfloat16(float(k)), dist)
        d32 = dist.astype(jnp.float32)
        wh = jnp.exp(d32 * -0.5)
        dx = pred_ref[pl.ds(base, BLK), 0:1] - predt[0:1, :]
        dy = pred_ref[pl.ds(base, BLK), 1:2] - predt[1:2, :]
        dz = pred_ref[pl.ds(base, BLK), 2:3] - predt[2:3, :]
        d2 = dx * dx + dy * dy + dz * dz + 1e-12
        wl = jnp.exp(-jnp.sqrt(d2))
        rr = wh - wl
        return acc + jnp.sum(rr * rr)

    acc = jax.lax.fori_loop(0, NBLK, block_step, jnp.zeros((), jnp.float32))
    mani_ref[...] = acc.reshape(1, 1)


def _run_dense(mt, x, gnn_W, gnn_b, mlp_W1, mlp_b1, mlp_W2, mlp_b2,
               batch, pos):
    f32 = jnp.float32
    bf16 = jnp.bfloat16
    pred, gf, ploss, mani = pl.pallas_call(
        _fused_body,
        out_shape=[
            jax.ShapeDtypeStruct((N, 3), f32),
            jax.ShapeDtypeStruct((G, D), f32),
            jax.ShapeDtypeStruct((1, 1), f32),
            jax.ShapeDtypeStruct((1, 1), f32),
        ],
        scratch_shapes=[
            pltpu.VMEM((N, D), f32),
            pltpu.VMEM((N, D), f32),
            pltpu.VMEM((N, N), bf16),
        ],
    )(mt, x, gnn_W, gnn_b, mlp_W1, mlp_b1.reshape(1, D), mlp_W2,
      mlp_b2.reshape(1, 3), batch.reshape(1, N), pos)
    return pred, gf, ploss.reshape(()), mani.reshape(())


_NS = 16
_NC = 2
_NW = _NS * _NC
_PROWS = 32
_NPASS = N // (_NW * _PROWS)
_ECH = 8192
_NECH = E // _ECH


def _edge_scatter_body(edge_ref, out_ref, src_v, dst_v, buf):
    c = jax.lax.axis_index("c")
    s = jax.lax.axis_index("s")
    wid = s * _NC + c

    zv = jnp.zeros((16,), jnp.float32)
    ones = jnp.ones((16,), jnp.float32)

    for p in range(_NPASS):
        base = wid * (_PROWS * _NPASS) + p * _PROWS

        def zrow(i, carry):
            buf[pl.ds(i * 16, 16)] = zv
            return carry

        jax.lax.fori_loop(0, _PROWS * N // 16, zrow, 0)

        for ec in range(_NECH):
            pltpu.sync_copy(edge_ref.at[0, pl.ds(ec * _ECH, _ECH)], src_v)
            pltpu.sync_copy(edge_ref.at[1, pl.ds(ec * _ECH, _ECH)], dst_v)

            def scan(g, carry):
                vs = src_v[pl.ds(g * 16, 16)]
                vd = dst_v[pl.ds(g * 16, 16)]
                m = (vd >= base) & (vd < base + _PROWS)
                fi = jnp.where(m, (vd - base) * N + vs, 0)
                plsc.addupdate_scatter(buf, [fi], ones, mask=m)
                return carry

            jax.lax.fori_loop(0, _ECH // 16, scan, 0)

        pltpu.sync_copy(buf, out_ref.at[pl.ds(base * N, _PROWS * N)])


@functools.partial(
    pl.kernel,
    out_type=jax.ShapeDtypeStruct((N * N,), jnp.float32),
    mesh=plsc.VectorSubcoreMesh(core_axis_name="c", subcore_axis_name="s"),
    compiler_params=pltpu.CompilerParams(needs_layout_passes=False),
    scratch_types=[
        pltpu.VMEM((_ECH,), jnp.int32),
        pltpu.VMEM((_ECH,), jnp.int32),
        pltpu.VMEM((_PROWS * N,), jnp.float32),
    ],
)
def _edge_scatter(edge_ref, out_ref, src_v, dst_v, buf):
    _edge_scatter_body(edge_ref, out_ref, src_v, dst_v, buf)


def kernel(x, pos, gnn_W, gnn_b, mlp_W1, mlp_b1, mlp_W2, mlp_b2,
           edge_index, batch):
    mt = _edge_scatter(edge_index).reshape(N, N)
    return _run_dense(mt, x, gnn_W, gnn_b, mlp_W1, mlp_b1, mlp_W2, mlp_b2,
                      batch, pos)


# speedup vs baseline: 3.2174x; 1.1313x over previous
import functools

import jax
import jax.numpy as jnp
from jax.experimental import pallas as pl
from jax.experimental.pallas import tpu as pltpu
from jax.experimental.pallas import tpu_sc as plsc

N = 2048
E = 32768
D = 128
L = 3
G = 16
BFS_ITERS = 6
BLK = 256
NBLK = N // BLK

_HIGH = jax.lax.Precision.HIGHEST


def _dot(a, b):
    return jnp.dot(a, b, precision=_HIGH, preferred_element_type=jnp.float32)


def _fused_body(mt_ref, x_ref, gw_ref, gb_ref, w1_ref, b1_ref, w2_ref,
                b2_ref, batch_ref, pos_ref,
                pred_ref, gf_ref, ploss_ref, mani_ref,
                h_ref, t_ref, a_ref):
    h_ref[...] = x_ref[...]
    for l in range(L):
        def blk(b, carry, l=l):
            rows = pl.ds(b * BLK, BLK)
            mtb = mt_ref[rows, :]
            if l == 0:
                a_ref[rows, :] = (mtb > 0.0).astype(jnp.bfloat16)
            aggb = _dot(mtb, h_ref[...])
            degb = jnp.sum(mtb, axis=1, keepdims=True) + 1.0
            t_ref[rows, :] = (h_ref[rows, :] + aggb) / degb
            return carry

        jax.lax.fori_loop(0, NBLK, blk, 0)
        hn = _dot(t_ref[...], gw_ref[l]) + gb_ref[l:l + 1, :]
        if l < L - 1:
            hn = jnp.maximum(hn, 0.0)
        h_ref[...] = hn
    h = h_ref[...]
    bb = jnp.broadcast_to(batch_ref[...], (G, N))
    gi = jax.lax.broadcasted_iota(jnp.int32, (G, N), 0)
    p = (bb == gi).astype(jnp.float32)
    counts = jnp.sum(p, axis=1, keepdims=True)
    gf_ref[...] = _dot(p, h) / jnp.maximum(counts, 1.0)
    t = jnp.maximum(_dot(h, w1_ref[...]) + b1_ref[...], 0.0)
    pred = _dot(t, w2_ref[...]) + b2_ref[...]
    pred_ref[...] = pred
    d = pred - pos_ref[...]
    ploss_ref[...] = (jnp.sum(d * d) * (1.0 / (N * 3))).reshape(1, 1)
    predt = jax.lax.transpose(pred, (1, 0))

    one = jnp.bfloat16(1.0)
    inf = jnp.bfloat16(jnp.inf)

    def build(b, carry):
        base = b * BLK
        ubt_b = jax.lax.transpose(a_ref[:, pl.ds(base, BLK)], (1, 0))
        a_ref[pl.ds(base, BLK), :] = jnp.maximum(
            a_ref[pl.ds(base, BLK), :], ubt_b)
        return carry

    jax.lax.fori_loop(0, NBLK, build, 0)
    a = a_ref[...]

    def block_step(b, acc):
        base = b * BLK
        ii = jax.lax.broadcasted_iota(jnp.int32, (BLK, N), 0) + base
        jj = jax.lax.broadcasted_iota(jnp.int32, (BLK, N), 1)
        eye = ii == jj
        ab = a_ref[pl.ds(base, BLK), :]
        r = jnp.where(eye, one, ab)
        dist = jnp.where(eye, jnp.bfloat16(0.0), jnp.where(ab > 0, one, inf))
        for k in range(2, BFS_ITERS + 1):
            prod = jnp.dot(r, a, preferred_element_type=jnp.float32)
            r = jnp.where(prod > 0.0, one, r)
            dist = jnp.where((r > 0) & (dist == inf),
                             jnp.bfloat16(float(k)), dist)
        d32 = dist.astype(jnp.float32)
        wh = jnp.exp(d32 * -0.5)
        dx = pred_ref[pl.ds(base, BLK), 0:1] - predt[0:1, :]
        dy = pred_ref[pl.ds(base, BLK), 1:2] - predt[1:2, :]
        dz = pred_ref[pl.ds(base, BLK), 2:3] - predt[2:3, :]
        d2 = dx * dx + dy * dy + dz * dz + 1e-12
        wl = jnp.exp(-jnp.sqrt(d2))
        rr = wh - wl
        return acc + jnp.sum(rr * rr)

    acc = jax.lax.fori_loop(0, NBLK, block_step, jnp.zeros((), jnp.float32))
    mani_ref[...] = acc.reshape(1, 1)


def _run_dense(mt, x, gnn_W, gnn_b, mlp_W1, mlp_b1, mlp_W2, mlp_b2,
               batch, pos):
    f32 = jnp.float32
    bf16 = jnp.bfloat16
    pred, gf, ploss, mani = pl.pallas_call(
        _fused_body,
        out_shape=[
            jax.ShapeDtypeStruct((N, 3), f32),
            jax.ShapeDtypeStruct((G, D), f32),
            jax.ShapeDtypeStruct((1, 1), f32),
            jax.ShapeDtypeStruct((1, 1), f32),
        ],
        scratch_shapes=[
            pltpu.VMEM((N, D), f32),
            pltpu.VMEM((N, D), f32),
            pltpu.VMEM((N, N), bf16),
        ],
    )(mt, x, gnn_W, gnn_b, mlp_W1, mlp_b1.reshape(1, D), mlp_W2,
      mlp_b2.reshape(1, 3), batch.reshape(1, N), pos)
    return pred, gf, ploss.reshape(()), mani.reshape(())


_NS = 16
_NC = 2
_NW = _NS * _NC
_PROWS = 32
_NPASS = N // (_NW * _PROWS)
_ECH = 8192
_NECH = E // _ECH


def _edge_scatter_body(edge_ref, out_ref, src_v, dst_v, buf):
    c = jax.lax.axis_index("c")
    s = jax.lax.axis_index("s")
    wid = s * _NC + c

    zv = jnp.zeros((16,), jnp.float32)
    ones = jnp.ones((16,), jnp.float32)

    for p in range(_NPASS):
        base = wid * (_PROWS * _NPASS) + p * _PROWS

        def zrow(i, carry):
            for u in range(8):
                buf[pl.ds(i * 128 + u * 16, 16)] = zv
            return carry

        jax.lax.fori_loop(0, _PROWS * N // 128, zrow, 0)

        for ec in range(_NECH):
            pltpu.sync_copy(edge_ref.at[0, pl.ds(ec * _ECH, _ECH)], src_v)
            pltpu.sync_copy(edge_ref.at[1, pl.ds(ec * _ECH, _ECH)], dst_v)

            def scan(g, carry):
                for u in range(4):
                    o = g * 64 + u * 16
                    vs = src_v[pl.ds(o, 16)]
                    vd = dst_v[pl.ds(o, 16)]
                    m = (vd >= base) & (vd < base + _PROWS)
                    fi = jnp.where(m, (vd - base) * N + vs, 0)
                    plsc.addupdate_scatter(buf, [fi], ones, mask=m)
                return carry

            jax.lax.fori_loop(0, _ECH // 64, scan, 0)

        pltpu.sync_copy(buf, out_ref.at[pl.ds(base * N, _PROWS * N)])


@functools.partial(
    pl.kernel,
    out_type=jax.ShapeDtypeStruct((N * N,), jnp.float32),
    mesh=plsc.VectorSubcoreMesh(core_axis_name="c", subcore_axis_name="s"),
    compiler_params=pltpu.CompilerParams(needs_layout_passes=False),
    scratch_types=[
        pltpu.VMEM((_ECH,), jnp.int32),
        pltpu.VMEM((_ECH,), jnp.int32),
        pltpu.VMEM((_PROWS * N,), jnp.float32),
    ],
)
def _edge_scatter(edge_ref, out_ref, src_v, dst_v, buf):
    _edge_scatter_body(edge_ref, out_ref, src_v, dst_v, buf)


def kernel(x, pos, gnn_W, gnn_b, mlp_W1, mlp_b1, mlp_W2, mlp_b2,
           edge_index, batch):
    mt = _edge_scatter(edge_index).reshape(N, N)
    return _run_dense(mt, x, gnn_W, gnn_b, mlp_W1, mlp_b1, mlp_W2, mlp_b2,
                      batch, pos)
